# Initial kernel scaffold; baseline (speedup 1.0000x reference)
import jax
import jax.numpy as jnp
from jax.experimental import pallas as pl


def kernel(x_content, edge_index, W1l, b1l, W1r, W2l, b2l, W2r, Wout, bout):
    raise NotImplementedError("write your pallas kernel here")


# speedup vs baseline: 2.8532x; 2.8532x over previous
import jax
import jax.numpy as jnp
from jax import lax
from jax.experimental import pallas as pl
from jax.experimental.pallas import tpu as pltpu
from jax.experimental.pallas import tpu_sc as plsc

N = 10000
NP = 10240
E = 320000
D = 128
CH = 128
NW = 32
EW = 10240
G = EW // CH
EP = EW * NW
EPC = EP // CH
RPT = NP // 16
BN = 1000

_MESH = plsc.VectorSubcoreMesh(core_axis_name="c", subcore_axis_name="s")


def _sc_agg_make():
    scratch = [
        pltpu.VMEM((G, CH), jnp.int32),
        pltpu.VMEM((G, CH), jnp.int32),
        pltpu.VMEM((CH, D), jnp.float32),
        pltpu.VMEM_SHARED((NP, D), jnp.float32),
        pltpu.SemaphoreType.DMA,
    ]

    def body(tbl_hbm, pk_hbm, part, idx_s, idx_d, rows, acc, sem):
        c = lax.axis_index("c")
        s = lax.axis_index("s")
        wid = c * 16 + s
        base = s * RPT

        pltpu.sync_copy(pk_hbm.at[pl.ds(wid * G, G)], idx_s)

        def fill(i, _):
            for k in range(D // 16):
                rows[i, pl.ds(k * 16, 16)] = jnp.zeros((16,), jnp.float32)
            return 0

        def unpack(i, _):
            for k in range(CH // 16):
                v = idx_s[i, pl.ds(k * 16, 16)]
                idx_d[i, pl.ds(k * 16, 16)] = lax.shift_right_logical(v, 16)
                idx_s[i, pl.ds(k * 16, 16)] = jnp.bitwise_and(v, 0xFFFF)
            return 0

        lax.fori_loop(0, CH, fill, 0)
        lax.fori_loop(0, G, unpack, 0)

        for k in range(RPT // CH):
            pltpu.sync_copy(rows, acc.at[pl.ds(base + k * CH, CH)])
        plsc.subcore_barrier()

        def chunk(j, _):
            pltpu.async_copy(tbl_hbm.at[idx_s.at[j]], rows, sem).wait()
            pltpu.sync_copy(rows, acc.at[idx_d.at[j]], add=True)
            return 0

        lax.fori_loop(0, G, chunk, 0)
        plsc.subcore_barrier()

        pltpu.sync_copy(acc.at[pl.ds(base, RPT)], part.at[c, pl.ds(base, RPT)])

    return pl.kernel(
        body,
        out_type=jax.ShapeDtypeStruct((2, NP, D), jnp.float32),
        mesh=_MESH,
        scratch_types=scratch,
    )


def _sc_deg_make():
    scratch = [
        pltpu.VMEM((G, CH), jnp.int32),
        pltpu.VMEM((CH, D), jnp.float32),
        pltpu.VMEM_SHARED((NP, D), jnp.float32),
    ]

    def body(pk_hbm, degp, idx_d, ones, dacc):
        c = lax.axis_index("c")
        s = lax.axis_index("s")
        wid = c * 16 + s
        base = s * RPT

        pltpu.sync_copy(pk_hbm.at[pl.ds(wid * G, G)], idx_d)

        def fill(val):
            def f(i, _):
                for k in range(D // 16):
                    ones[i, pl.ds(k * 16, 16)] = jnp.full((16,), val,
                                                          jnp.float32)
                return 0
            return f

        def unpack(i, _):
            for k in range(CH // 16):
                v = idx_d[i, pl.ds(k * 16, 16)]
                idx_d[i, pl.ds(k * 16, 16)] = lax.shift_right_logical(v, 16)
            return 0

        lax.fori_loop(0, CH, fill(0.0), 0)
        lax.fori_loop(0, G, unpack, 0)
        for k in range(RPT // CH):
            pltpu.sync_copy(ones, dacc.at[pl.ds(base + k * CH, CH)])
        lax.fori_loop(0, CH, fill(1.0), 0)
        plsc.subcore_barrier()

        def chunk(j, _):
            pltpu.sync_copy(ones, dacc.at[idx_d.at[j]], add=True)
            return 0

        lax.fori_loop(0, G, chunk, 0)
        plsc.subcore_barrier()
        pltpu.sync_copy(dacc.at[pl.ds(base, RPT)],
                        degp.at[c, pl.ds(base, RPT)])

    return pl.kernel(
        body,
        out_type=jax.ShapeDtypeStruct((2, NP, D), jnp.float32),
        mesh=_MESH,
        scratch_types=scratch,
    )


_sc_agg = _sc_agg_make()
_sc_deg = _sc_deg_make()

_CONTRACT = (((1,), (1,)), ((), ()))


def _tc_pre(x, W1l, W1r, b1l):
    def body(x_ref, wl_ref, wr_ref, b_ref, p_ref, q_ref):
        xb = x_ref[...]
        p_ref[...] = lax.dot_general(xb, wl_ref[...], _CONTRACT,
                                     preferred_element_type=jnp.float32)
        q_ref[...] = lax.dot_general(xb, wr_ref[...], _CONTRACT,
                                     preferred_element_type=jnp.float32) + b_ref[...]

    return pl.pallas_call(
        body,
        grid=(N // BN,),
        in_specs=[
            pl.BlockSpec((BN, D), lambda i: (i, 0)),
            pl.BlockSpec((D, D), lambda i: (0, 0)),
            pl.BlockSpec((D, D), lambda i: (0, 0)),
            pl.BlockSpec((1, D), lambda i: (0, 0)),
        ],
        out_specs=[pl.BlockSpec((BN, D), lambda i: (i, 0))] * 2,
        out_shape=[jax.ShapeDtypeStruct((N, D), jnp.float32)] * 2,
    )(x, W1l, W1r, b1l.reshape(1, D))


def _tc_mid(part, degp, q1, W2l, W2r, b2l):
    def body(pa_ref, dg_ref, q_ref, wl_ref, wr_ref, b_ref, p2_ref, q2_ref):
        ps = pa_ref[0] + pa_ref[1]
        dg = dg_ref[0, :, 0:1] + dg_ref[1, :, 0:1]
        inv = 1.0 / jnp.maximum(dg, 1.0)
        h = jnp.maximum(ps * inv + q_ref[...], 0.0)
        p2_ref[...] = lax.dot_general(h, wl_ref[...], _CONTRACT,
                                      preferred_element_type=jnp.float32)
        q2_ref[...] = lax.dot_general(h, wr_ref[...], _CONTRACT,
                                      preferred_element_type=jnp.float32) + b_ref[...]

    return pl.pallas_call(
        body,
        grid=(N // BN,),
        in_specs=[
            pl.BlockSpec((2, BN, D), lambda i: (0, i, 0)),
            pl.BlockSpec((2, BN, D), lambda i: (0, i, 0)),
            pl.BlockSpec((BN, D), lambda i: (i, 0)),
            pl.BlockSpec((D, D), lambda i: (0, 0)),
            pl.BlockSpec((D, D), lambda i: (0, 0)),
            pl.BlockSpec((1, D), lambda i: (0, 0)),
        ],
        out_specs=[pl.BlockSpec((BN, D), lambda i: (i, 0))] * 2,
        out_shape=[jax.ShapeDtypeStruct((N, D), jnp.float32)] * 2,
    )(part, degp, q1, W2l, W2r, b2l.reshape(1, D))


def _tc_out(part, degp, q2, Wout, bout):
    C = Wout.shape[0]

    def body(pa_ref, dg_ref, q_ref, w_ref, b_ref, o_ref):
        ps = pa_ref[0] + pa_ref[1]
        dg = dg_ref[0, :, 0:1] + dg_ref[1, :, 0:1]
        inv = 1.0 / jnp.maximum(dg, 1.0)
        h = jnp.maximum(ps * inv + q_ref[...], 0.0)
        o_ref[...] = lax.dot_general(h, w_ref[...], _CONTRACT,
                                     preferred_element_type=jnp.float32) + b_ref[...]

    return pl.pallas_call(
        body,
        grid=(N // BN,),
        in_specs=[
            pl.BlockSpec((2, BN, D), lambda i: (0, i, 0)),
            pl.BlockSpec((2, BN, D), lambda i: (0, i, 0)),
            pl.BlockSpec((BN, D), lambda i: (i, 0)),
            pl.BlockSpec((C, D), lambda i: (0, 0)),
            pl.BlockSpec((1, C), lambda i: (0, 0)),
        ],
        out_specs=pl.BlockSpec((BN, C), lambda i: (i, 0)),
        out_shape=jax.ShapeDtypeStruct((N, C), jnp.float32),
    )(part, degp, q2, Wout, bout.reshape(1, C))


def kernel(x_content, edge_index, W1l, b1l, W1r, W2l, b2l, W2r, Wout, bout):
    ei = edge_index.astype(jnp.int32)
    pad = EP - E
    src_p = jnp.concatenate([ei[0], jnp.zeros((pad,), jnp.int32)])
    dst_p = jnp.concatenate(
        [ei[1], N + (jnp.arange(pad, dtype=jnp.int32) % (NP - N))])
    packed = jnp.bitwise_or(src_p, dst_p << 16).reshape(EPC, CH)

    degp = _sc_deg(packed)
    p1, q1 = _tc_pre(x_content, W1l, W1r, b1l)
    part1 = _sc_agg(p1, packed)
    p2, q2 = _tc_mid(part1, degp, q1, W2l, W2r, b2l)
    part2 = _sc_agg(p2, packed)
    return _tc_out(part2, degp, q2, Wout, bout)


# speedup vs baseline: 10.0261x; 3.5140x over previous
import jax
import jax.numpy as jnp
from jax import lax
from jax.experimental import pallas as pl
from jax.experimental.pallas import tpu as pltpu
from jax.experimental.pallas import tpu_sc as plsc

N = 10000
NP = 10240
E = 320000
D = 128
CH = 80
NW = 32
EW = E // NW
G = EW // CH

RPT = NP // 16
BN = 1000

_MESH = plsc.VectorSubcoreMesh(core_axis_name="c", subcore_axis_name="s")


def _sc_agg_make():
    scratch = [
        pltpu.VMEM((EW,), jnp.int32),
        pltpu.VMEM((G, CH), jnp.int32),
        pltpu.VMEM((2, CH, D), jnp.float32),
        pltpu.VMEM_SHARED((NP, D), jnp.float32),
        pltpu.SemaphoreType.DMA((2,)),
    ]

    def body(tbl_hbm, pk_hbm, part, idx_s, idx_d, rows, acc, sem):
        c = lax.axis_index("c")
        s = lax.axis_index("s")
        wid = c * 16 + s
        base = s * RPT

        pltpu.sync_copy(pk_hbm.at[wid], idx_d)

        def fill(i, _):
            for k in range(D // 16):
                rows[0, i, pl.ds(k * 16, 16)] = jnp.zeros((16,), jnp.float32)
            return 0

        def unpack(i, _):
            for k in range(CH // 16):
                v = idx_d[i, pl.ds(k * 16, 16)]
                idx_s[pl.ds(i * CH + k * 16, 16)] = jnp.bitwise_and(v, 0xFFFF)
                idx_d[i, pl.ds(k * 16, 16)] = lax.shift_right_logical(v, 16)
            return 0

        lax.fori_loop(0, CH, fill, 0)
        lax.fori_loop(0, G, unpack, 0)

        for k in range(RPT // CH):
            pltpu.sync_copy(rows.at[0], acc.at[pl.ds(base + k * CH, CH)])
        plsc.subcore_barrier()

        def fire(j, b):
            pltpu.async_copy(tbl_hbm.at[idx_s.at[pl.ds(j * CH, CH)]],
                             rows.at[b], sem.at[b])

        def drain(j, b):
            pltpu.make_async_copy(tbl_hbm.at[idx_s.at[pl.ds(j * CH, CH)]],
                                  rows.at[b], sem.at[b]).wait()

        fire(0, 0)

        def chunk(j, _):
            b = lax.rem(j, 2)

            @pl.when(j + 1 < G)
            def _():
                fire(j + 1, 1 - b)

            drain(j, b)
            pltpu.sync_copy(rows.at[b], acc.at[idx_d.at[j]], add=True)
            return 0

        lax.fori_loop(0, G, chunk, 0)
        plsc.subcore_barrier()

        pltpu.sync_copy(acc.at[pl.ds(base, RPT)], part.at[c, pl.ds(base, RPT)])

    return pl.kernel(
        body,
        out_type=jax.ShapeDtypeStruct((2, NP, D), jnp.float32),
        mesh=_MESH,
        scratch_types=scratch,
    )


def _sc_deg_make():
    scratch = [
        pltpu.VMEM((G, CH), jnp.int32),
        pltpu.VMEM((CH, D), jnp.float32),
        pltpu.VMEM_SHARED((NP, D), jnp.float32),
    ]

    def body(pk_hbm, degp, idx_d, ones, dacc):
        c = lax.axis_index("c")
        s = lax.axis_index("s")
        wid = c * 16 + s
        base = s * RPT

        pltpu.sync_copy(pk_hbm.at[wid], idx_d)

        def fill(val):
            def f(i, _):
                for k in range(D // 16):
                    ones[i, pl.ds(k * 16, 16)] = jnp.full((16,), val,
                                                          jnp.float32)
                return 0
            return f

        def unpack(i, _):
            for k in range(CH // 16):
                v = idx_d[i, pl.ds(k * 16, 16)]
                idx_d[i, pl.ds(k * 16, 16)] = lax.shift_right_logical(v, 16)
            return 0

        lax.fori_loop(0, CH, fill(0.0), 0)
        lax.fori_loop(0, G, unpack, 0)
        for k in range(RPT // CH):
            pltpu.sync_copy(ones, dacc.at[pl.ds(base + k * CH, CH)])
        lax.fori_loop(0, CH, fill(1.0), 0)
        plsc.subcore_barrier()

        def chunk(j, _):
            pltpu.sync_copy(ones, dacc.at[idx_d.at[j]], add=True)
            return 0

        lax.fori_loop(0, G, chunk, 0)
        plsc.subcore_barrier()
        pltpu.sync_copy(dacc.at[pl.ds(base, RPT)],
                        degp.at[c, pl.ds(base, RPT)])

    return pl.kernel(
        body,
        out_type=jax.ShapeDtypeStruct((2, NP, D), jnp.float32),
        mesh=_MESH,
        scratch_types=scratch,
    )


_sc_agg = _sc_agg_make()
_sc_deg = _sc_deg_make()

_CONTRACT = (((1,), (1,)), ((), ()))


def _tc_pre(x, W1l, W1r, b1l):
    def body(x_ref, wl_ref, wr_ref, b_ref, p_ref, q_ref):
        xb = x_ref[...]
        p_ref[...] = lax.dot_general(xb, wl_ref[...], _CONTRACT,
                                     preferred_element_type=jnp.float32)
        q_ref[...] = lax.dot_general(xb, wr_ref[...], _CONTRACT,
                                     preferred_element_type=jnp.float32) + b_ref[...]

    return pl.pallas_call(
        body,
        grid=(N // BN,),
        in_specs=[
            pl.BlockSpec((BN, D), lambda i: (i, 0)),
            pl.BlockSpec((D, D), lambda i: (0, 0)),
            pl.BlockSpec((D, D), lambda i: (0, 0)),
            pl.BlockSpec((1, D), lambda i: (0, 0)),
        ],
        out_specs=[pl.BlockSpec((BN, D), lambda i: (i, 0))] * 2,
        out_shape=[jax.ShapeDtypeStruct((N, D), jnp.float32)] * 2,
    )(x, W1l, W1r, b1l.reshape(1, D))


def _tc_mid(part, degp, q1, W2l, W2r, b2l):
    def body(pa_ref, dg_ref, q_ref, wl_ref, wr_ref, b_ref, p2_ref, q2_ref):
        ps = pa_ref[0] + pa_ref[1]
        dg = dg_ref[0, :, 0:1] + dg_ref[1, :, 0:1]
        inv = 1.0 / jnp.maximum(dg, 1.0)
        h = jnp.maximum(ps * inv + q_ref[...], 0.0)
        p2_ref[...] = lax.dot_general(h, wl_ref[...], _CONTRACT,
                                      preferred_element_type=jnp.float32)
        q2_ref[...] = lax.dot_general(h, wr_ref[...], _CONTRACT,
                                      preferred_element_type=jnp.float32) + b_ref[...]

    return pl.pallas_call(
        body,
        grid=(N // BN,),
        in_specs=[
            pl.BlockSpec((2, BN, D), lambda i: (0, i, 0)),
            pl.BlockSpec((2, BN, D), lambda i: (0, i, 0)),
            pl.BlockSpec((BN, D), lambda i: (i, 0)),
            pl.BlockSpec((D, D), lambda i: (0, 0)),
            pl.BlockSpec((D, D), lambda i: (0, 0)),
            pl.BlockSpec((1, D), lambda i: (0, 0)),
        ],
        out_specs=[pl.BlockSpec((BN, D), lambda i: (i, 0))] * 2,
        out_shape=[jax.ShapeDtypeStruct((N, D), jnp.float32)] * 2,
    )(part, degp, q1, W2l, W2r, b2l.reshape(1, D))


def _tc_out(part, degp, q2, Wout, bout):
    C = Wout.shape[0]

    def body(pa_ref, dg_ref, q_ref, w_ref, b_ref, o_ref):
        ps = pa_ref[0] + pa_ref[1]
        dg = dg_ref[0, :, 0:1] + dg_ref[1, :, 0:1]
        inv = 1.0 / jnp.maximum(dg, 1.0)
        h = jnp.maximum(ps * inv + q_ref[...], 0.0)
        o_ref[...] = lax.dot_general(h, w_ref[...], _CONTRACT,
                                     preferred_element_type=jnp.float32) + b_ref[...]

    return pl.pallas_call(
        body,
        grid=(N // BN,),
        in_specs=[
            pl.BlockSpec((2, BN, D), lambda i: (0, i, 0)),
            pl.BlockSpec((2, BN, D), lambda i: (0, i, 0)),
            pl.BlockSpec((BN, D), lambda i: (i, 0)),
            pl.BlockSpec((C, D), lambda i: (0, 0)),
            pl.BlockSpec((1, C), lambda i: (0, 0)),
        ],
        out_specs=pl.BlockSpec((BN, C), lambda i: (i, 0)),
        out_shape=jax.ShapeDtypeStruct((N, C), jnp.float32),
    )(part, degp, q2, Wout, bout.reshape(1, C))


def kernel(x_content, edge_index, W1l, b1l, W1r, W2l, b2l, W2r, Wout, bout):
    ei = edge_index.astype(jnp.int32)
    packed = jnp.bitwise_or(ei[0], ei[1] << 16).reshape(NW, G, CH)

    degp = _sc_deg(packed)
    p1, q1 = _tc_pre(x_content, W1l, W1r, b1l)
    part1 = _sc_agg(p1, packed)
    p2, q2 = _tc_mid(part1, degp, q1, W2l, W2r, b2l)
    part2 = _sc_agg(p2, packed)
    return _tc_out(part2, degp, q2, Wout, bout)
